# baseline (speedup 1.0000x reference)
import functools

import jax
import jax.numpy as jnp
from jax import lax
from jax.experimental import pallas as pl
from jax.experimental.pallas import tpu as pltpu
from jax.experimental.pallas import tpu_sc as plsc

NC, NS = 2, 16
NW = NC * NS
TPU_LANES = 16
TOK = 128
EMB = 64
NTOK = 16384
NK = 50
UNITS = NK * (NTOK // TOK)
UPW = UNITS // NW
TT_BLOCKS = NTOK // TOK


RB = 3906
NUM_ROWS = 1000000


@jax.jit
def _repack(wt):
    mesh = plsc.VectorSubcoreMesh(core_axis_name="c", subcore_axis_name="s")

    @functools.partial(
        pl.kernel,
        out_type=jax.ShapeDtypeStruct((500000, 128), jnp.float32),
        mesh=mesh,
        scratch_types=[
            pltpu.VMEM((EMB, 256), jnp.float32),
            pltpu.VMEM((EMB, 256), jnp.float32),
            pltpu.VMEM((TOK, TOK), jnp.float32),
            pltpu.VMEM((TOK, TOK), jnp.float32),
            pltpu.SemaphoreType.DMA,
            pltpu.SemaphoreType.DMA,
            pltpu.SemaphoreType.DMA,
            pltpu.SemaphoreType.DMA,
        ],
        compiler_params=pltpu.CompilerParams(needs_layout_passes=False),
    )
    def k(wt_hbm, out_hbm, i0, i1, o0, o1, si0, si1, so0, so1):
        wid = lax.axis_index("s") * NC + lax.axis_index("c")
        base = RB // NW
        extra = RB - base * NW
        start = wid * base + lax.min(wid, extra)
        end = start + base + jnp.where(wid < extra, 1, 0)

        def ifire(b, ibuf, sem):
            pltpu.async_copy(wt_hbm.at[:, pl.ds(b * 256, 256)], ibuf, sem)

        def iwait(ibuf, sem):
            pltpu.make_async_copy(wt_hbm.at[:, pl.ds(0, 256)], ibuf, sem).wait()

        def ofireb(b, obuf, sem):
            pltpu.async_copy(obuf, out_hbm.at[pl.ds(b * TOK, TOK)], sem)

        def owaitb(obuf, sem):
            pltpu.make_async_copy(obuf, out_hbm.at[pl.ds(0, TOK)], sem).wait()

        lvec = lax.iota(jnp.int32, TPU_LANES)

        def trans(ibuf, obuf, nig):
            @plsc.parallel_loop(0, nig, unroll=2)
            def _(ig):
                ivec = lvec + ig * TPU_LANES
                ibase = ivec << 6
                for j in range(EMB):
                    jl = (lvec + j) & (EMB - 1)
                    val = plsc.load_gather(ibuf, [jl, ivec])
                    f = ibase + jl
                    plsc.store_scatter(obuf, [f >> 7, f & (TOK - 1)], val)

        ifire(start, i0, si0)

        @pl.loop(0, 124, step=2)
        def _(s):
            b = start + s

            @pl.when(b < end)
            def _():
                @pl.when(b + 1 < end)
                def _():
                    ifire(b + 1, i1, si1)
                iwait(i0, si0)

                @pl.when(s >= 2)
                def _():
                    owaitb(o0, so0)
                trans(i0, o0, 16)
                ofireb(b, o0, so0)

            @pl.when(b + 1 < end)
            def _():
                @pl.when(b + 2 < end)
                def _():
                    ifire(b + 2, i0, si0)
                iwait(i1, si1)

                @pl.when(s >= 2)
                def _():
                    owaitb(o1, so1)
                trans(i1, o1, 16)
                ofireb(b + 1, o1, so1)

        owaitb(o0, so0)
        owaitb(o1, so1)

    return k(wt)


@jax.jit
def _lookup(idx_flat, table2):
    mesh = plsc.VectorSubcoreMesh(core_axis_name="c", subcore_axis_name="s")

    @functools.partial(
        pl.kernel,
        out_type=jax.ShapeDtypeStruct((NK, EMB, NTOK), jnp.float32),
        mesh=mesh,
        scratch_types=[
            pltpu.VMEM((UPW * TOK,), jnp.int32),
            pltpu.VMEM((TOK, TOK), jnp.float32),
            pltpu.VMEM((TOK, TOK), jnp.float32),
            pltpu.VMEM((EMB, TOK), jnp.float32),
            pltpu.VMEM((EMB, TOK), jnp.float32),
            pltpu.VMEM((TOK,), jnp.int32),
            pltpu.VMEM((TOK,), jnp.int32),
            pltpu.VMEM((TOK,), jnp.int32),
            pltpu.VMEM((TOK,), jnp.int32),
            pltpu.SemaphoreType.DMA,
            pltpu.SemaphoreType.DMA,
            pltpu.SemaphoreType.DMA,
            pltpu.SemaphoreType.DMA,
        ],
        compiler_params=pltpu.CompilerParams(needs_layout_passes=False),
    )
    def k(idx_hbm, tab_hbm, out_hbm,
          idx_v, g0, g1, o0, o1, h0, h1, p0, p1, sg0, sg1, so0, so1):
        wid = lax.axis_index("s") * NC + lax.axis_index("c")
        u0 = wid * UPW
        pltpu.sync_copy(idx_hbm.at[pl.ds(u0 * TOK, UPW * TOK)], idx_v)

        def prep(lu, hv, pv):
            for c in range(TOK // TPU_LANES):
                v = idx_v[pl.ds(lu * TOK + c * TPU_LANES, TPU_LANES)]
                hv[pl.ds(c * TPU_LANES, TPU_LANES)] = lax.shift_right_logical(v, 1)
                pv[pl.ds(c * TPU_LANES, TPU_LANES)] = (v & 1) << 6

        def gfire(hv, g, sem):
            pltpu.async_copy(tab_hbm.at[hv], g, sem)

        def gwait(g, sem):
            pltpu.make_async_copy(tab_hbm.at[h0], g, sem).wait()

        def transpose(g, pv, obuf):
            lvec = lax.iota(jnp.int32, TPU_LANES)

            @plsc.parallel_loop(0, TOK // TPU_LANES, unroll=4)
            def _(tg):
                tvec = lvec + tg * TPU_LANES
                pvec = pv[pl.ds(tg * TPU_LANES, TPU_LANES)]
                for j in range(EMB):
                    jl = (lvec + j) & (EMB - 1)
                    val = plsc.load_gather(g, [tvec, pvec + jl])
                    plsc.store_scatter(obuf, [jl, tvec], val)

        def ofire(u, obuf, sem):
            kk = lax.div(u, TT_BLOCKS)
            tt = lax.rem(u, TT_BLOCKS)
            pltpu.async_copy(obuf, out_hbm.at[kk, :, pl.ds(tt * TOK, TOK)], sem)

        def owait(obuf, sem):
            pltpu.make_async_copy(obuf, out_hbm.at[0, :, pl.ds(0, TOK)], sem).wait()

        prep(0, h0, p0)
        gfire(h0, g0, sg0)

        @pl.loop(0, UPW, step=2)
        def _(lu):
            prep(lu + 1, h1, p1)
            gfire(h1, g1, sg1)
            gwait(g0, sg0)

            @pl.when(lu >= 2)
            def _():
                owait(o0, so0)
            transpose(g0, p0, o0)
            ofire(u0 + lu, o0, so0)

            @pl.when(lu + 2 < UPW)
            def _():
                prep(lu + 2, h0, p0)
                gfire(h0, g0, sg0)
            gwait(g1, sg1)

            @pl.when(lu >= 2)
            def _():
                owait(o1, so1)
            transpose(g1, p1, o1)
            ofire(u0 + lu + 1, o1, so1)

        owait(o0, so0)
        owait(o1, so1)

    return k(idx_flat, table2)


def kernel(token_ids, weight):
    idx_flat = jnp.transpose(token_ids).reshape(-1).astype(jnp.int32)
    table2 = _repack(jnp.transpose(weight))
    tail = lax.slice(weight, (RB * 256, 0), (NUM_ROWS, EMB)).reshape(32, 2 * EMB)
    table2 = lax.dynamic_update_slice(table2, tail, (RB * TOK, 0))
    o3 = _lookup(idx_flat, table2)
    return jnp.transpose(o3, (2, 0, 1))


# speedup vs baseline: 1.0450x; 1.0450x over previous
import functools

import jax
import jax.numpy as jnp
from jax import lax
from jax.experimental import pallas as pl
from jax.experimental.pallas import tpu as pltpu
from jax.experimental.pallas import tpu_sc as plsc

NC, NS = 2, 16
NW = NC * NS
TPU_LANES = 16
TOK = 128
EMB = 64
NTOK = 16384
NK = 50
UNITS = NK * (NTOK // TOK)
UPW = UNITS // NW
TT_BLOCKS = NTOK // TOK


RB = 3906
NUM_ROWS = 1000000


@jax.jit
def _repack(wt):
    mesh = plsc.VectorSubcoreMesh(core_axis_name="c", subcore_axis_name="s")

    @functools.partial(
        pl.kernel,
        out_type=jax.ShapeDtypeStruct((500000, 128), jnp.float32),
        mesh=mesh,
        scratch_types=[
            pltpu.VMEM((EMB, 256), jnp.float32),
            pltpu.VMEM((EMB, 256), jnp.float32),
            pltpu.VMEM((TOK, TOK), jnp.float32),
            pltpu.VMEM((TOK, TOK), jnp.float32),
            pltpu.SemaphoreType.DMA,
            pltpu.SemaphoreType.DMA,
            pltpu.SemaphoreType.DMA,
            pltpu.SemaphoreType.DMA,
        ],
        compiler_params=pltpu.CompilerParams(needs_layout_passes=False),
    )
    def k(wt_hbm, out_hbm, i0, i1, o0, o1, si0, si1, so0, so1):
        wid = lax.axis_index("s") * NC + lax.axis_index("c")
        base = RB // NW
        extra = RB - base * NW
        start = wid * base + lax.min(wid, extra)
        end = start + base + jnp.where(wid < extra, 1, 0)

        def ifire(b, ibuf, sem):
            pltpu.async_copy(wt_hbm.at[:, pl.ds(b * 256, 256)], ibuf, sem)

        def iwait(ibuf, sem):
            pltpu.make_async_copy(wt_hbm.at[:, pl.ds(0, 256)], ibuf, sem).wait()

        def ofireb(b, obuf, sem):
            pltpu.async_copy(obuf, out_hbm.at[pl.ds(b * TOK, TOK)], sem)

        def owaitb(obuf, sem):
            pltpu.make_async_copy(obuf, out_hbm.at[pl.ds(0, TOK)], sem).wait()

        lvec = lax.iota(jnp.int32, TPU_LANES)

        def trans(ibuf, obuf, nig):
            @plsc.parallel_loop(0, nig, unroll=2)
            def _(ig):
                ivec = lvec + ig * TPU_LANES
                ibase = ivec << 6
                for j in range(EMB):
                    jl = (lvec + j) & (EMB - 1)
                    val = plsc.load_gather(ibuf, [jl, ivec])
                    f = ibase + jl
                    plsc.store_scatter(obuf, [f >> 7, f & (TOK - 1)], val)

        ifire(start, i0, si0)

        @pl.loop(0, 124, step=2)
        def _(s):
            b = start + s

            @pl.when(b < end)
            def _():
                @pl.when(b + 1 < end)
                def _():
                    ifire(b + 1, i1, si1)
                iwait(i0, si0)

                @pl.when(s >= 2)
                def _():
                    owaitb(o0, so0)
                trans(i0, o0, 16)
                ofireb(b, o0, so0)

            @pl.when(b + 1 < end)
            def _():
                @pl.when(b + 2 < end)
                def _():
                    ifire(b + 2, i0, si0)
                iwait(i1, si1)

                @pl.when(s >= 2)
                def _():
                    owaitb(o1, so1)
                trans(i1, o1, 16)
                ofireb(b + 1, o1, so1)

        owaitb(o0, so0)
        owaitb(o1, so1)

    return k(wt)


@jax.jit
def _lookup(idx_flat, table2):
    mesh = plsc.VectorSubcoreMesh(core_axis_name="c", subcore_axis_name="s")

    @functools.partial(
        pl.kernel,
        out_type=jax.ShapeDtypeStruct((NK, EMB, NTOK), jnp.float32),
        mesh=mesh,
        scratch_types=[
            pltpu.VMEM((UPW * TOK,), jnp.int32),
            pltpu.VMEM((TOK, TOK), jnp.float32),
            pltpu.VMEM((TOK, TOK), jnp.float32),
            pltpu.VMEM((EMB, TOK), jnp.float32),
            pltpu.VMEM((EMB, TOK), jnp.float32),
            pltpu.VMEM((TOK,), jnp.int32),
            pltpu.VMEM((TOK,), jnp.int32),
            pltpu.VMEM((TOK,), jnp.int32),
            pltpu.VMEM((TOK,), jnp.int32),
            pltpu.SemaphoreType.DMA,
            pltpu.SemaphoreType.DMA,
            pltpu.SemaphoreType.DMA,
            pltpu.SemaphoreType.DMA,
        ],
        compiler_params=pltpu.CompilerParams(needs_layout_passes=False),
    )
    def k(idx_hbm, tab_hbm, out_hbm,
          idx_v, g0, g1, o0, o1, h0, h1, p0, p1, sg0, sg1, so0, so1):
        wid = lax.axis_index("s") * NC + lax.axis_index("c")
        u0 = wid * UPW
        pltpu.sync_copy(idx_hbm.at[pl.ds(u0 * TOK, UPW * TOK)], idx_v)

        def prep(lu, hv, pv):
            for c in range(TOK // TPU_LANES):
                v = idx_v[pl.ds(lu * TOK + c * TPU_LANES, TPU_LANES)]
                hv[pl.ds(c * TPU_LANES, TPU_LANES)] = lax.shift_right_logical(v, 1)
                pv[pl.ds(c * TPU_LANES, TPU_LANES)] = (v & 1) << 6

        def gfire(hv, g, sem):
            pltpu.async_copy(tab_hbm.at[hv], g, sem)

        def gwait(g, sem):
            pltpu.make_async_copy(tab_hbm.at[h0], g, sem).wait()

        def transpose(g, pv, obuf):
            lvec = lax.iota(jnp.int32, TPU_LANES)

            @plsc.parallel_loop(0, TOK // TPU_LANES, unroll=2)
            def _(tg):
                tvec = lvec + tg * TPU_LANES
                pvec = pv[pl.ds(tg * TPU_LANES, TPU_LANES)]
                for j in range(EMB):
                    jl = (lvec + j) & (EMB - 1)
                    val = plsc.load_gather(g, [tvec, pvec + jl])
                    plsc.store_scatter(obuf, [jl, tvec], val)

        def ofire(u, obuf, sem):
            kk = lax.div(u, TT_BLOCKS)
            tt = lax.rem(u, TT_BLOCKS)
            pltpu.async_copy(obuf, out_hbm.at[kk, :, pl.ds(tt * TOK, TOK)], sem)

        def owait(obuf, sem):
            pltpu.make_async_copy(obuf, out_hbm.at[0, :, pl.ds(0, TOK)], sem).wait()

        prep(0, h0, p0)
        gfire(h0, g0, sg0)

        @pl.loop(0, UPW, step=2)
        def _(lu):
            prep(lu + 1, h1, p1)
            gfire(h1, g1, sg1)
            gwait(g0, sg0)

            @pl.when(lu >= 2)
            def _():
                owait(o0, so0)
            transpose(g0, p0, o0)
            ofire(u0 + lu, o0, so0)

            @pl.when(lu + 2 < UPW)
            def _():
                prep(lu + 2, h0, p0)
                gfire(h0, g0, sg0)
            gwait(g1, sg1)

            @pl.when(lu >= 2)
            def _():
                owait(o1, so1)
            transpose(g1, p1, o1)
            ofire(u0 + lu + 1, o1, so1)

        owait(o0, so0)
        owait(o1, so1)

    return k(idx_flat, table2)


def kernel(token_ids, weight):
    idx_flat = jnp.transpose(token_ids).reshape(-1).astype(jnp.int32)
    table2 = _repack(jnp.transpose(weight))
    tail = lax.slice(weight, (RB * 256, 0), (NUM_ROWS, EMB)).reshape(32, 2 * EMB)
    table2 = lax.dynamic_update_slice(table2, tail, (RB * TOK, 0))
    o3 = _lookup(idx_flat, table2)
    return jnp.transpose(o3, (2, 0, 1))
